# Initial kernel scaffold; baseline (speedup 1.0000x reference)
import functools

import jax
import jax.numpy as jnp
from jax import lax
from jax.experimental import pallas as pl
from jax.experimental.pallas import tpu as pltpu
from jax.experimental.pallas import tpu_sc as plsc

_DIM = 64
_TOT = 16384 * 26
_NW = 32
_PER_W = _TOT // _NW
_CH = 128
_NCH = _PER_W // _CH


def _make_kernel():
    mesh = plsc.VectorSubcoreMesh(core_axis_name="c", subcore_axis_name="s")

    @functools.partial(
        pl.kernel,
        mesh=mesh,
        out_type=jax.ShapeDtypeStruct((_TOT, _DIM), jnp.float32),
        scratch_types=[
            pltpu.VMEM((_NCH, _CH), jnp.int32),
            pltpu.VMEM((_CH, _DIM), jnp.float32),
            pltpu.SemaphoreType.DMA,
        ],
    )
    def gather_kernel(idx_hbm, table_hbm, out_hbm, idx_v, rows_v, sem):
        wid = lax.axis_index("s") * 2 + lax.axis_index("c")
        pltpu.sync_copy(idx_hbm.at[pl.ds(wid * _NCH, _NCH)], idx_v)
        base = wid * _PER_W

        def step(g, carry):
            pltpu.async_copy(table_hbm.at[idx_v.at[g]], rows_v, sem).wait()
            pltpu.sync_copy(rows_v, out_hbm.at[pl.ds(base + g * _CH, _CH)])
            return carry

        lax.fori_loop(0, _NCH, step, 0)

    return gather_kernel


_KERNEL = _make_kernel()


def kernel(indices, weight):
    flat = indices.reshape(_TOT // _CH, _CH)
    out = _KERNEL(flat, weight)
    return out.reshape(indices.shape + (weight.shape[-1],))


# speedup vs baseline: 5.9217x; 5.9217x over previous
import functools

import jax
import jax.numpy as jnp
from jax import lax
from jax.experimental import pallas as pl
from jax.experimental.pallas import tpu as pltpu
from jax.experimental.pallas import tpu_sc as plsc

_DIM = 64
_TOT = 16384 * 26
_NW = 32
_PER_W = _TOT // _NW
_CH = 128
_NCH = _PER_W // _CH


def _make_kernel():
    mesh = plsc.VectorSubcoreMesh(core_axis_name="c", subcore_axis_name="s")

    @functools.partial(
        pl.kernel,
        mesh=mesh,
        out_type=jax.ShapeDtypeStruct((_TOT, _DIM), jnp.float32),
        compiler_params=pltpu.CompilerParams(use_tc_tiling_on_sc=False),
        scratch_types=[
            pltpu.VMEM((_NCH, _CH), jnp.int32),
            pltpu.VMEM((_CH, _DIM), jnp.float32),
            pltpu.SemaphoreType.DMA,
        ],
    )
    def gather_kernel(idx_hbm, table_hbm, out_hbm, idx_v, rows_v, sem):
        wid = lax.axis_index("s") * 2 + lax.axis_index("c")
        pltpu.sync_copy(idx_hbm.at[pl.ds(wid * _NCH, _NCH)], idx_v)
        base = wid * _PER_W

        def step(g, carry):
            pltpu.async_copy(table_hbm.at[idx_v.at[g]], rows_v, sem).wait()
            pltpu.sync_copy(rows_v, out_hbm.at[pl.ds(base + g * _CH, _CH)])
            return carry

        lax.fori_loop(0, _NCH, step, 0)

    return gather_kernel


_KERNEL = _make_kernel()


def kernel(indices, weight):
    flat = indices.reshape(_TOT // _CH, _CH)
    out = _KERNEL(flat, weight)
    return out.reshape(indices.shape + (weight.shape[-1],))


# speedup vs baseline: 6.3878x; 1.0787x over previous
import functools

import jax
import jax.numpy as jnp
from jax import lax
from jax.experimental import pallas as pl
from jax.experimental.pallas import tpu as pltpu
from jax.experimental.pallas import tpu_sc as plsc

_DIM = 64
_TOT = 16384 * 26
_NW = 32
_PER_W = _TOT // _NW
_IR = 128
_NIR = _PER_W // _IR
_CH = 256
_NCH = _PER_W // _CH
_NSLOT = 4


def _make_kernel():
    mesh = plsc.VectorSubcoreMesh(core_axis_name="c", subcore_axis_name="s")

    @functools.partial(
        pl.kernel,
        mesh=mesh,
        out_type=jax.ShapeDtypeStruct((_TOT, _DIM), jnp.float32),
        compiler_params=pltpu.CompilerParams(use_tc_tiling_on_sc=False),
        scratch_types=[
            pltpu.VMEM((_NIR, _IR), jnp.int32),
            [pltpu.VMEM((_CH, _DIM), jnp.float32)] * _NSLOT,
            [pltpu.SemaphoreType.DMA] * _NSLOT,
            [pltpu.SemaphoreType.DMA] * _NSLOT,
        ],
    )
    def gather_kernel(idx_hbm, table_hbm, out_hbm, idx_v, rows, gsem, ssem):
        wid = lax.axis_index("s") * 2 + lax.axis_index("c")
        pltpu.sync_copy(idx_hbm.at[pl.ds(wid * _NIR, _NIR)], idx_v)
        base = wid * _PER_W

        def gfire(ci, s):
            for k in range(_CH // _IR):
                pltpu.async_copy(
                    table_hbm.at[idx_v.at[ci * (_CH // _IR) + k]],
                    rows[s].at[pl.ds(k * _IR, _IR)],
                    gsem[s],
                )

        def gwait(s):
            pltpu.make_async_copy(
                out_hbm.at[pl.ds(0, _CH)], rows[s], gsem[s]
            ).wait()

        def sfire(ci, s):
            pltpu.async_copy(
                rows[s], out_hbm.at[pl.ds(base + ci * _CH, _CH)], ssem[s]
            )

        def swait(s):
            pltpu.make_async_copy(
                rows[s], out_hbm.at[pl.ds(base, _CH)], ssem[s]
            ).wait()

        for s in range(_NSLOT):
            gfire(s, s)
        gwait(0)
        sfire(0, 0)

        def outer(j, carry):
            for b in range(_NSLOT):
                i = _NSLOT + j * _NSLOT + b
                swait(b)
                gfire(i, b)
                b2 = (b + 1) % _NSLOT
                gwait(b2)
                sfire(i - (_NSLOT - 1), b2)
            return carry

        lax.fori_loop(0, (_NCH - _NSLOT) // _NSLOT, outer, 0)

        for e in range(_NSLOT - 1):
            i = _NCH + e
            b2 = (i + 1) % _NSLOT
            gwait(b2)
            sfire(i - (_NSLOT - 1), b2)
        for s in range(_NSLOT):
            swait(s)

    return gather_kernel


_KERNEL = _make_kernel()


def kernel(indices, weight):
    flat = indices.reshape(_TOT // _IR, _IR)
    out = _KERNEL(flat, weight)
    return out.reshape(indices.shape + (weight.shape[-1],))
